# Initial kernel scaffold; baseline (speedup 1.0000x reference)
import jax
import jax.numpy as jnp
from jax.experimental import pallas as pl


def kernel(x, edge_index, W, b):
    raise NotImplementedError("write your pallas kernel here")


# speedup vs baseline: 22.1557x; 22.1557x over previous
import functools

import jax
import jax.numpy as jnp
from jax import lax
from jax.experimental import pallas as pl
from jax.experimental.pallas import tpu as pltpu
import jax.experimental.pallas.tpu_sc as plsc

N = 10000
E = 160000
D = 256
DH = 128
NC = 2
NS = 16
K = 125
ROWS_PER_TILE = 624
TAIL_ROWS = N - NS * ROWS_PER_TILE
RB = 2000
GRID = N // RB


def _deg_body(dst_hbm, ones_hbm, zeros_hbm, deg_out, dst_v, ones_v, deg_sh,
              dsem):
    c = lax.axis_index("c")
    s = lax.axis_index("s")
    nch = dst_hbm.shape[2]
    pltpu.sync_copy(dst_hbm.at[c, s], dst_v)
    pltpu.sync_copy(ones_hbm, ones_v)

    @pl.when(s == 0)
    def _():
        pltpu.sync_copy(zeros_hbm, deg_sh)

    plsc.subcore_barrier()

    WIN = 4

    for g in range(WIN):
        pltpu.async_copy(ones_v, deg_sh.at[dst_v.at[g]], dsem, add=True)

    def body(g, carry):
        pltpu.make_async_copy(ones_v, deg_sh.at[dst_v.at[0]], dsem).wait()

        @pl.when(g + WIN < nch)
        def _():
            pltpu.async_copy(ones_v, deg_sh.at[dst_v.at[g + WIN]], dsem,
                             add=True)

        return carry

    lax.fori_loop(0, nch, body, 0, unroll=False)
    plsc.subcore_barrier()

    @pl.when(s == 0)
    def _():
        pltpu.sync_copy(deg_sh, deg_out.at[c])


def _deg_call(dst_r, ones, zeros):
    mesh = plsc.VectorSubcoreMesh(core_axis_name="c", subcore_axis_name="s")
    nch = dst_r.shape[2]
    return pl.kernel(
        _deg_body,
        out_type=jax.ShapeDtypeStruct((NC, N), jnp.float32),
        mesh=mesh,
        scratch_types=[
            pltpu.VMEM((nch, K), jnp.int32),
            pltpu.VMEM((K,), jnp.float32),
            pltpu.VMEM_SHARED((N,), jnp.float32),
            pltpu.SemaphoreType.DMA,
        ],
    )(dst_r, ones, zeros)


def _agg_body(hp_hbm, src_hbm, dst_hbm, raw_out, src_v, dst_v, rows_v, acc_sh,
              gsem, ssem, isem):
    c = lax.axis_index("c")
    s = lax.axis_index("s")
    nch = src_hbm.shape[1]
    pltpu.sync_copy(src_hbm.at[s], src_v)
    pltpu.sync_copy(dst_hbm.at[s, 0], dst_v.at[0])
    pltpu.sync_copy(hp_hbm.at[c, pl.ds(s * ROWS_PER_TILE, ROWS_PER_TILE)],
                    acc_sh.at[pl.ds(s * ROWS_PER_TILE, ROWS_PER_TILE)])

    @pl.when(s == 0)
    def _():
        pltpu.sync_copy(hp_hbm.at[c, pl.ds(NS * ROWS_PER_TILE, TAIL_ROWS)],
                        acc_sh.at[pl.ds(NS * ROWS_PER_TILE, TAIL_ROWS)])

    plsc.subcore_barrier()

    niter = nch // 2
    pltpu.async_copy(hp_hbm.at[c].at[src_v.at[0]], rows_v.at[0], gsem)

    def body(i, carry):
        g0 = 2 * i
        i1 = pltpu.async_copy(dst_hbm.at[s, g0 + 1], dst_v.at[1], isem)
        pltpu.make_async_copy(hp_hbm.at[c].at[src_v.at[g0]], rows_v.at[0],
                              gsem).wait()
        s0 = pltpu.async_copy(rows_v.at[0], acc_sh.at[dst_v.at[0]], ssem,
                              add=True)
        g1 = pltpu.async_copy(hp_hbm.at[c].at[src_v.at[g0 + 1]], rows_v.at[1],
                              gsem)
        g1.wait()
        i1.wait()
        s1 = pltpu.async_copy(rows_v.at[1], acc_sh.at[dst_v.at[1]], ssem,
                              add=True)
        s0.wait()

        @pl.when(i + 1 < niter)
        def _():
            pltpu.async_copy(hp_hbm.at[c].at[src_v.at[g0 + 2]], rows_v.at[0],
                             gsem)
            pltpu.async_copy(dst_hbm.at[s, g0 + 2], dst_v.at[0], isem)

        s1.wait()

        @pl.when(i + 1 < niter)
        def _():
            pltpu.make_async_copy(dst_hbm.at[s, g0 + 2], dst_v.at[0],
                                  isem).wait()

        return carry

    lax.fori_loop(0, niter, body, 0, unroll=False)
    plsc.subcore_barrier()
    pltpu.sync_copy(acc_sh.at[pl.ds(s * ROWS_PER_TILE, ROWS_PER_TILE)],
                    raw_out.at[c, pl.ds(s * ROWS_PER_TILE, ROWS_PER_TILE)])

    @pl.when(s == 0)
    def _():
        pltpu.sync_copy(acc_sh.at[pl.ds(NS * ROWS_PER_TILE, TAIL_ROWS)],
                        raw_out.at[c, pl.ds(NS * ROWS_PER_TILE, TAIL_ROWS)])


def _agg_call(hp, src_r, dst_r):
    mesh = plsc.VectorSubcoreMesh(core_axis_name="c", subcore_axis_name="s")
    nch = src_r.shape[1]
    return pl.kernel(
        _agg_body,
        out_type=jax.ShapeDtypeStruct((NC, N, DH), jnp.float32),
        mesh=mesh,
        scratch_types=[
            pltpu.VMEM((nch, K), jnp.int32),
            pltpu.VMEM((2, K), jnp.int32),
            pltpu.VMEM((2, K, DH), jnp.float32),
            pltpu.VMEM_SHARED((N, DH), jnp.float32),
            pltpu.SemaphoreType.DMA,
            pltpu.SemaphoreType.DMA,
            pltpu.SemaphoreType.DMA,
        ],
    )(hp, src_r, dst_r)


def _mm_body(x_ref, w_ref, o_ref):
    o_ref[...] = jnp.dot(x_ref[...], w_ref[...],
                         preferred_element_type=jnp.float32)


def _mm_call(x, W):
    return pl.pallas_call(
        _mm_body,
        grid=(GRID,),
        in_specs=[
            pl.BlockSpec((RB, D), lambda i: (i, 0)),
            pl.BlockSpec((D, D), lambda i: (0, 0)),
        ],
        out_specs=pl.BlockSpec((RB, D), lambda i: (i, 0)),
        out_shape=jax.ShapeDtypeStruct((N, D), jnp.float32),
    )(x, W)


def _dis_from(deg_ref):
    dval = deg_ref[...]
    d = dval[0, 0, :] + dval[0, 1, :] + 1.0
    return lax.rsqrt(d)


def _scale_body(h_ref, deg_ref, hp_ref):
    dis = _dis_from(deg_ref)
    hs = h_ref[...] * dis[:, None]
    hp_ref[0] = hs[:, :DH]
    hp_ref[1] = hs[:, DH:]


def _scale_call(h, deg_r):
    return pl.pallas_call(
        _scale_body,
        grid=(GRID,),
        in_specs=[
            pl.BlockSpec((RB, D), lambda i: (i, 0)),
            pl.BlockSpec((1, NC, RB), lambda i: (i, 0, 0)),
        ],
        out_specs=pl.BlockSpec((NC, RB, DH), lambda i: (0, i, 0)),
        out_shape=jax.ShapeDtypeStruct((NC, N, DH), jnp.float32),
    )(h, deg_r)


def _final_body(raw_ref, deg_ref, b_ref, o_ref):
    dis = _dis_from(deg_ref)
    r = jnp.concatenate([raw_ref[0], raw_ref[1]], axis=-1)
    o_ref[...] = dis[:, None] * r + b_ref[...]


def _final_call(raw, deg_r, b2):
    return pl.pallas_call(
        _final_body,
        grid=(GRID,),
        in_specs=[
            pl.BlockSpec((NC, RB, DH), lambda i: (0, i, 0)),
            pl.BlockSpec((1, NC, RB), lambda i: (i, 0, 0)),
            pl.BlockSpec((1, D), lambda i: (0, 0)),
        ],
        out_specs=pl.BlockSpec((RB, D), lambda i: (i, 0)),
        out_shape=jax.ShapeDtypeStruct((N, D), jnp.float32),
    )(raw, deg_r, b2)


def kernel(x, edge_index, W, b):
    src = edge_index[0].astype(jnp.int32)
    dst = edge_index[1].astype(jnp.int32)
    dst_a = dst.reshape(NC, NS, E // (NC * NS * K), K)
    src_c = src.reshape(NS, E // (NS * K), K)
    dst_c = dst.reshape(NS, E // (NS * K), K)
    ones = jnp.ones((K,), jnp.float32)
    zeros = jnp.zeros((N,), jnp.float32)

    deg_p = _deg_call(dst_a, ones, zeros)
    h = _mm_call(x, W)
    deg_r = deg_p.reshape(NC, GRID, RB).transpose(1, 0, 2)
    hp = _scale_call(h, deg_r)
    raw = _agg_call(hp, src_c, dst_c)
    return _final_call(raw, deg_r, b.reshape(1, D))


# speedup vs baseline: 27.5157x; 1.2419x over previous
import functools

import jax
import jax.numpy as jnp
from jax import lax
from jax.experimental import pallas as pl
from jax.experimental.pallas import tpu as pltpu
import jax.experimental.pallas.tpu_sc as plsc

N = 10000
E = 160000
D = 256
DH = 128
NC = 2
NS = 16
K = 80
DEPTH = 3
KD = 125
ROWS_PER_TILE = 624
TAIL_ROWS = N - NS * ROWS_PER_TILE
RB = 2000
GRID = N // RB


def _deg_body(dst_hbm, ones_hbm, zeros_hbm, deg_out, dst_v, ones_v, deg_sh,
              dsem):
    c = lax.axis_index("c")
    s = lax.axis_index("s")
    nch = dst_hbm.shape[2]
    pltpu.sync_copy(dst_hbm.at[c, s], dst_v)
    pltpu.sync_copy(ones_hbm, ones_v)

    @pl.when(s == 0)
    def _():
        pltpu.sync_copy(zeros_hbm, deg_sh)

    plsc.subcore_barrier()

    WIN = 4

    for g in range(WIN):
        pltpu.async_copy(ones_v, deg_sh.at[dst_v.at[g]], dsem, add=True)

    def body(g, carry):
        pltpu.make_async_copy(ones_v, deg_sh.at[dst_v.at[0]], dsem).wait()

        @pl.when(g + WIN < nch)
        def _():
            pltpu.async_copy(ones_v, deg_sh.at[dst_v.at[g + WIN]], dsem,
                             add=True)

        return carry

    lax.fori_loop(0, nch, body, 0, unroll=False)
    plsc.subcore_barrier()

    @pl.when(s == 0)
    def _():
        pltpu.sync_copy(deg_sh, deg_out.at[c])


def _deg_call(dst_r, ones, zeros):
    mesh = plsc.VectorSubcoreMesh(core_axis_name="c", subcore_axis_name="s")
    nch = dst_r.shape[2]
    return pl.kernel(
        _deg_body,
        out_type=jax.ShapeDtypeStruct((NC, N), jnp.float32),
        mesh=mesh,
        scratch_types=[
            pltpu.VMEM((nch, KD), jnp.int32),
            pltpu.VMEM((KD,), jnp.float32),
            pltpu.VMEM_SHARED((N,), jnp.float32),
            pltpu.SemaphoreType.DMA,
        ],
    )(dst_r, ones, zeros)


def _agg_body(hp_hbm, src_hbm, dst_hbm, raw_out, src_v, dst_v, rows_v, acc_sh,
              gsem, ssem, isem):
    c = lax.axis_index("c")
    s = lax.axis_index("s")
    nch = dst_hbm.shape[1]
    pltpu.sync_copy(src_hbm.at[s], src_v)
    pltpu.sync_copy(hp_hbm.at[c, pl.ds(s * ROWS_PER_TILE, ROWS_PER_TILE)],
                    acc_sh.at[pl.ds(s * ROWS_PER_TILE, ROWS_PER_TILE)])

    @pl.when(s == 0)
    def _():
        pltpu.sync_copy(hp_hbm.at[c, pl.ds(NS * ROWS_PER_TILE, TAIL_ROWS)],
                        acc_sh.at[pl.ds(NS * ROWS_PER_TILE, TAIL_ROWS)])

    plsc.subcore_barrier()

    def g_idx(g):
        return src_v.at[pl.ds(pl.multiple_of(g * K, 8), K)]

    def drain_gather():
        pltpu.make_async_copy(hp_hbm.at[c].at[g_idx(0)], rows_v.at[0],
                              gsem).wait()

    def drain_idx():
        pltpu.make_async_copy(dst_hbm.at[s, 0], dst_v.at[0], isem).wait()

    def drain_scatter():
        pltpu.make_async_copy(rows_v.at[0], acc_sh.at[dst_v.at[0]],
                              ssem).wait()

    for j in range(DEPTH):
        pltpu.async_copy(dst_hbm.at[s, j], dst_v.at[j], isem)
        pltpu.async_copy(hp_hbm.at[c].at[g_idx(j)], rows_v.at[j], gsem)

    def body(g, carry):
        b = lax.rem(g, DEPTH)
        drain_gather()
        drain_idx()
        pltpu.async_copy(rows_v.at[b], acc_sh.at[dst_v.at[b]], ssem, add=True)

        @pl.when(g >= 1)
        def _():
            drain_scatter()

        @pl.when((g >= 1) & (g + DEPTH - 1 < nch))
        def _():
            gn = g + DEPTH - 1
            bn = lax.rem(gn, DEPTH)
            pltpu.async_copy(dst_hbm.at[s, gn], dst_v.at[bn], isem)
            pltpu.async_copy(hp_hbm.at[c].at[g_idx(gn)], rows_v.at[bn], gsem)

        return carry

    lax.fori_loop(0, nch, body, 0, unroll=False)
    drain_scatter()
    plsc.subcore_barrier()
    pltpu.sync_copy(acc_sh.at[pl.ds(s * ROWS_PER_TILE, ROWS_PER_TILE)],
                    raw_out.at[c, pl.ds(s * ROWS_PER_TILE, ROWS_PER_TILE)])

    @pl.when(s == 0)
    def _():
        pltpu.sync_copy(acc_sh.at[pl.ds(NS * ROWS_PER_TILE, TAIL_ROWS)],
                        raw_out.at[c, pl.ds(NS * ROWS_PER_TILE, TAIL_ROWS)])


def _agg_call(hp, src_r, dst_r):
    mesh = plsc.VectorSubcoreMesh(core_axis_name="c", subcore_axis_name="s")
    epw = src_r.shape[1]
    return pl.kernel(
        _agg_body,
        out_type=jax.ShapeDtypeStruct((NC, N, DH), jnp.float32),
        mesh=mesh,
        scratch_types=[
            pltpu.VMEM((epw,), jnp.int32),
            pltpu.VMEM((DEPTH, K), jnp.int32),
            pltpu.VMEM((DEPTH, K, DH), jnp.float32),
            pltpu.VMEM_SHARED((N, DH), jnp.float32),
            pltpu.SemaphoreType.DMA,
            pltpu.SemaphoreType.DMA,
            pltpu.SemaphoreType.DMA,
        ],
    )(hp, src_r, dst_r)


def _dis_from(deg_ref):
    dval = deg_ref[...]
    d = dval[0, 0, :] + dval[0, 1, :] + 1.0
    return lax.rsqrt(d)


def _mmscale_body(x_ref, w_ref, deg_ref, hp_ref):
    dis = _dis_from(deg_ref)
    h = jnp.dot(x_ref[...], w_ref[...], preferred_element_type=jnp.float32)
    hs = h * dis[:, None]
    hp_ref[0] = hs[:, :DH]
    hp_ref[1] = hs[:, DH:]


def _mmscale_call(x, W, deg_r):
    return pl.pallas_call(
        _mmscale_body,
        grid=(GRID,),
        in_specs=[
            pl.BlockSpec((RB, D), lambda i: (i, 0)),
            pl.BlockSpec((D, D), lambda i: (0, 0)),
            pl.BlockSpec((1, NC, RB), lambda i: (i, 0, 0)),
        ],
        out_specs=pl.BlockSpec((NC, RB, DH), lambda i: (0, i, 0)),
        out_shape=jax.ShapeDtypeStruct((NC, N, DH), jnp.float32),
    )(x, W, deg_r)


def _final_body(raw_ref, deg_ref, b_ref, o_ref):
    dis = _dis_from(deg_ref)
    r = jnp.concatenate([raw_ref[0], raw_ref[1]], axis=-1)
    o_ref[...] = dis[:, None] * r + b_ref[...]


def _final_call(raw, deg_r, b2):
    return pl.pallas_call(
        _final_body,
        grid=(GRID,),
        in_specs=[
            pl.BlockSpec((NC, RB, DH), lambda i: (0, i, 0)),
            pl.BlockSpec((1, NC, RB), lambda i: (i, 0, 0)),
            pl.BlockSpec((1, D), lambda i: (0, 0)),
        ],
        out_specs=pl.BlockSpec((RB, D), lambda i: (i, 0)),
        out_shape=jax.ShapeDtypeStruct((N, D), jnp.float32),
    )(raw, deg_r, b2)


def kernel(x, edge_index, W, b):
    src = edge_index[0].astype(jnp.int32)
    dst = edge_index[1].astype(jnp.int32)
    dst_a = dst.reshape(NC, NS, E // (NC * NS * KD), KD)
    src_c = src.reshape(NS, E // NS)
    dst_c = dst.reshape(NS, E // (NS * K), K)
    ones = jnp.ones((KD,), jnp.float32)
    zeros = jnp.zeros((N,), jnp.float32)

    deg_p = _deg_call(dst_a, ones, zeros)
    deg_r = deg_p.reshape(NC, GRID, RB).transpose(1, 0, 2)
    hp = _mmscale_call(x, W, deg_r)
    raw = _agg_call(hp, src_c, dst_c)
    return _final_call(raw, deg_r, b.reshape(1, D))


# speedup vs baseline: 28.0154x; 1.0182x over previous
import functools

import jax
import jax.numpy as jnp
from jax import lax
from jax.experimental import pallas as pl
from jax.experimental.pallas import tpu as pltpu
import jax.experimental.pallas.tpu_sc as plsc

N = 10000
E = 160000
D = 256
DH = 128
NC = 2
NS = 16
K = 80
DEPTH = 3
KD = 125
ROWS_PER_TILE = 624
TAIL_ROWS = N - NS * ROWS_PER_TILE
RB = 2000
GRID = N // RB


def _deg_body(dst_hbm, ones_hbm, zeros_hbm, deg_out, dst_v, ones_v, deg_sh,
              dsem):
    c = lax.axis_index("c")
    s = lax.axis_index("s")
    nch = dst_hbm.shape[2]
    pltpu.sync_copy(dst_hbm.at[c, s], dst_v)
    pltpu.sync_copy(ones_hbm, ones_v)

    @pl.when(s == 0)
    def _():
        pltpu.sync_copy(zeros_hbm, deg_sh)

    plsc.subcore_barrier()

    WIN = 4

    for g in range(WIN):
        pltpu.async_copy(ones_v, deg_sh.at[dst_v.at[g]], dsem, add=True)

    def body(g, carry):
        pltpu.make_async_copy(ones_v, deg_sh.at[dst_v.at[0]], dsem).wait()

        @pl.when(g + WIN < nch)
        def _():
            pltpu.async_copy(ones_v, deg_sh.at[dst_v.at[g + WIN]], dsem,
                             add=True)

        return carry

    lax.fori_loop(0, nch, body, 0, unroll=False)
    plsc.subcore_barrier()

    @pl.when(s == 0)
    def _():
        pltpu.sync_copy(deg_sh, deg_out.at[c])


def _deg_call(dst_r, ones, zeros):
    mesh = plsc.VectorSubcoreMesh(core_axis_name="c", subcore_axis_name="s")
    nch = dst_r.shape[2]
    return pl.kernel(
        _deg_body,
        out_type=jax.ShapeDtypeStruct((NC, N), jnp.float32),
        mesh=mesh,
        scratch_types=[
            pltpu.VMEM((nch, KD), jnp.int32),
            pltpu.VMEM((KD,), jnp.float32),
            pltpu.VMEM_SHARED((N,), jnp.float32),
            pltpu.SemaphoreType.DMA,
        ],
    )(dst_r, ones, zeros)


def _agg_body(hp_hbm, src_hbm, dst_hbm, raw_out, src_v, dst_v, rows_v, acc_sh,
              gsem, ssem, isem, asem):
    c = lax.axis_index("c")
    s = lax.axis_index("s")
    nch = dst_hbm.shape[1]
    init = pltpu.async_copy(
        hp_hbm.at[c, pl.ds(s * ROWS_PER_TILE, ROWS_PER_TILE)],
        acc_sh.at[pl.ds(s * ROWS_PER_TILE, ROWS_PER_TILE)], asem)

    @pl.when(s == 0)
    def _():
        pltpu.async_copy(hp_hbm.at[c, pl.ds(NS * ROWS_PER_TILE, TAIL_ROWS)],
                         acc_sh.at[pl.ds(NS * ROWS_PER_TILE, TAIL_ROWS)],
                         asem)

    pltpu.sync_copy(src_hbm.at[s], src_v)

    def g_idx(g):
        return src_v.at[pl.ds(pl.multiple_of(g * K, 8), K)]

    def drain_gather():
        pltpu.make_async_copy(hp_hbm.at[c].at[g_idx(0)], rows_v.at[0],
                              gsem).wait()

    def drain_idx():
        pltpu.make_async_copy(dst_hbm.at[s, 0], dst_v.at[0], isem).wait()

    def drain_scatter():
        pltpu.make_async_copy(rows_v.at[0], acc_sh.at[dst_v.at[0]],
                              ssem).wait()

    for j in range(DEPTH):
        pltpu.async_copy(dst_hbm.at[s, j], dst_v.at[j], isem)
        pltpu.async_copy(hp_hbm.at[c].at[g_idx(j)], rows_v.at[j], gsem)

    init.wait()

    @pl.when(s == 0)
    def _():
        pltpu.make_async_copy(
            hp_hbm.at[c, pl.ds(NS * ROWS_PER_TILE, TAIL_ROWS)],
            acc_sh.at[pl.ds(NS * ROWS_PER_TILE, TAIL_ROWS)], asem).wait()

    plsc.subcore_barrier()

    def body(g, carry):
        b = lax.rem(g, DEPTH)
        drain_gather()
        drain_idx()
        pltpu.async_copy(rows_v.at[b], acc_sh.at[dst_v.at[b]], ssem, add=True)

        @pl.when(g >= 1)
        def _():
            drain_scatter()

        @pl.when((g >= 1) & (g + DEPTH - 1 < nch))
        def _():
            gn = g + DEPTH - 1
            bn = lax.rem(gn, DEPTH)
            pltpu.async_copy(dst_hbm.at[s, gn], dst_v.at[bn], isem)
            pltpu.async_copy(hp_hbm.at[c].at[g_idx(gn)], rows_v.at[bn], gsem)

        return carry

    lax.fori_loop(0, nch, body, 0, unroll=False)
    drain_scatter()
    plsc.subcore_barrier()
    pltpu.sync_copy(acc_sh.at[pl.ds(s * ROWS_PER_TILE, ROWS_PER_TILE)],
                    raw_out.at[c, pl.ds(s * ROWS_PER_TILE, ROWS_PER_TILE)])

    @pl.when(s == 0)
    def _():
        pltpu.sync_copy(acc_sh.at[pl.ds(NS * ROWS_PER_TILE, TAIL_ROWS)],
                        raw_out.at[c, pl.ds(NS * ROWS_PER_TILE, TAIL_ROWS)])


def _agg_call(hp, src_r, dst_r):
    mesh = plsc.VectorSubcoreMesh(core_axis_name="c", subcore_axis_name="s")
    epw = src_r.shape[1]
    return pl.kernel(
        _agg_body,
        out_type=jax.ShapeDtypeStruct((NC, N, DH), jnp.float32),
        mesh=mesh,
        scratch_types=[
            pltpu.VMEM((epw,), jnp.int32),
            pltpu.VMEM((DEPTH, K), jnp.int32),
            pltpu.VMEM((DEPTH, K, DH), jnp.float32),
            pltpu.VMEM_SHARED((N, DH), jnp.float32),
            pltpu.SemaphoreType.DMA,
            pltpu.SemaphoreType.DMA,
            pltpu.SemaphoreType.DMA,
            pltpu.SemaphoreType.DMA,
        ],
    )(hp, src_r, dst_r)


def _dis_from(deg_ref):
    dval = deg_ref[...]
    d = dval[0, 0, :] + dval[0, 1, :] + 1.0
    return lax.rsqrt(d)


def _mmscale_body(x_ref, w_ref, deg_ref, hp_ref):
    dis = _dis_from(deg_ref)
    h = jnp.dot(x_ref[...], w_ref[...], preferred_element_type=jnp.float32)
    hs = h * dis[:, None]
    hp_ref[0] = hs[:, :DH]
    hp_ref[1] = hs[:, DH:]


def _mmscale_call(x, W, deg_r):
    return pl.pallas_call(
        _mmscale_body,
        grid=(GRID,),
        in_specs=[
            pl.BlockSpec((RB, D), lambda i: (i, 0)),
            pl.BlockSpec((D, D), lambda i: (0, 0)),
            pl.BlockSpec((1, NC, RB), lambda i: (i, 0, 0)),
        ],
        out_specs=pl.BlockSpec((NC, RB, DH), lambda i: (0, i, 0)),
        out_shape=jax.ShapeDtypeStruct((NC, N, DH), jnp.float32),
    )(x, W, deg_r)


def _final_body(raw_ref, deg_ref, b_ref, o_ref):
    dis = _dis_from(deg_ref)
    r = jnp.concatenate([raw_ref[0], raw_ref[1]], axis=-1)
    o_ref[...] = dis[:, None] * r + b_ref[...]


def _final_call(raw, deg_r, b2):
    return pl.pallas_call(
        _final_body,
        grid=(GRID,),
        in_specs=[
            pl.BlockSpec((NC, RB, DH), lambda i: (0, i, 0)),
            pl.BlockSpec((1, NC, RB), lambda i: (i, 0, 0)),
            pl.BlockSpec((1, D), lambda i: (0, 0)),
        ],
        out_specs=pl.BlockSpec((RB, D), lambda i: (i, 0)),
        out_shape=jax.ShapeDtypeStruct((N, D), jnp.float32),
    )(raw, deg_r, b2)


def kernel(x, edge_index, W, b):
    src = edge_index[0].astype(jnp.int32)
    dst = edge_index[1].astype(jnp.int32)
    dst_a = dst.reshape(NC, NS, E // (NC * NS * KD), KD)
    src_c = src.reshape(NS, E // NS)
    dst_c = dst.reshape(NS, E // (NS * K), K)
    ones = jnp.ones((KD,), jnp.float32)
    zeros = jnp.zeros((N,), jnp.float32)

    deg_p = _deg_call(dst_a, ones, zeros)
    deg_r = deg_p.reshape(NC, GRID, RB).transpose(1, 0, 2)
    hp = _mmscale_call(x, W, deg_r)
    raw = _agg_call(hp, src_c, dst_c)
    return _final_call(raw, deg_r, b.reshape(1, D))


# speedup vs baseline: 28.5301x; 1.0184x over previous
import functools

import jax
import jax.numpy as jnp
from jax import lax
from jax.experimental import pallas as pl
from jax.experimental.pallas import tpu as pltpu
import jax.experimental.pallas.tpu_sc as plsc

N = 10000
E = 160000
D = 256
DH = 128
NC = 2
NS = 16
K = 80
DEPTH = 4
IDEPTH = DEPTH + 1
KD = 125
ROWS_PER_TILE = 624
TAIL_ROWS = N - NS * ROWS_PER_TILE
RB = 2000
GRID = N // RB


def _deg_body(dst_hbm, ones_hbm, zeros_hbm, deg_out, dst_v, ones_v, deg_sh,
              dsem):
    c = lax.axis_index("c")
    s = lax.axis_index("s")
    nch = dst_hbm.shape[2]
    pltpu.sync_copy(dst_hbm.at[c, s], dst_v)
    pltpu.sync_copy(ones_hbm, ones_v)

    @pl.when(s == 0)
    def _():
        pltpu.sync_copy(zeros_hbm, deg_sh)

    plsc.subcore_barrier()

    WIN = 4

    for g in range(WIN):
        pltpu.async_copy(ones_v, deg_sh.at[dst_v.at[g]], dsem, add=True)

    def body(g, carry):
        pltpu.make_async_copy(ones_v, deg_sh.at[dst_v.at[0]], dsem).wait()

        @pl.when(g + WIN < nch)
        def _():
            pltpu.async_copy(ones_v, deg_sh.at[dst_v.at[g + WIN]], dsem,
                             add=True)

        return carry

    lax.fori_loop(0, nch, body, 0, unroll=False)
    plsc.subcore_barrier()

    @pl.when(s == 0)
    def _():
        pltpu.sync_copy(deg_sh, deg_out.at[c])


def _deg_call(dst_r, ones, zeros):
    mesh = plsc.VectorSubcoreMesh(core_axis_name="c", subcore_axis_name="s")
    nch = dst_r.shape[2]
    return pl.kernel(
        _deg_body,
        out_type=jax.ShapeDtypeStruct((NC, N), jnp.float32),
        mesh=mesh,
        scratch_types=[
            pltpu.VMEM((nch, KD), jnp.int32),
            pltpu.VMEM((KD,), jnp.float32),
            pltpu.VMEM_SHARED((N,), jnp.float32),
            pltpu.SemaphoreType.DMA,
        ],
    )(dst_r, ones, zeros)


def _agg_body(hp_hbm, ei_hbm, raw_out, idx_v, rows_v, acc_sh,
              gsem, ssem, isem, asem):
    c = lax.axis_index("c")
    s = lax.axis_index("s")
    nch = ei_hbm.shape[1]
    init = pltpu.async_copy(
        hp_hbm.at[c, pl.ds(s * ROWS_PER_TILE, ROWS_PER_TILE)],
        acc_sh.at[pl.ds(s * ROWS_PER_TILE, ROWS_PER_TILE)], asem)

    @pl.when(s == 0)
    def _():
        pltpu.async_copy(hp_hbm.at[c, pl.ds(NS * ROWS_PER_TILE, TAIL_ROWS)],
                         acc_sh.at[pl.ds(NS * ROWS_PER_TILE, TAIL_ROWS)],
                         asem)

    def drain_gather():
        pltpu.make_async_copy(hp_hbm.at[c].at[idx_v.at[0, 0]], rows_v.at[0],
                              gsem).wait()

    def drain_idx():
        pltpu.make_async_copy(ei_hbm.at[s, 0], idx_v.at[0], isem).wait()

    def drain_scatter():
        pltpu.make_async_copy(rows_v.at[0], acc_sh.at[idx_v.at[0, 1]],
                              ssem).wait()

    for j in range(DEPTH):
        pltpu.async_copy(ei_hbm.at[s, j], idx_v.at[j], isem)
    for j in range(DEPTH - 1):
        drain_idx()
        pltpu.async_copy(hp_hbm.at[c].at[idx_v.at[j, 0]], rows_v.at[j], gsem)

    init.wait()

    @pl.when(s == 0)
    def _():
        pltpu.make_async_copy(
            hp_hbm.at[c, pl.ds(NS * ROWS_PER_TILE, TAIL_ROWS)],
            acc_sh.at[pl.ds(NS * ROWS_PER_TILE, TAIL_ROWS)], asem).wait()

    plsc.subcore_barrier()

    def body(g, carry):
        b = lax.rem(g, DEPTH)
        bi = lax.rem(g, IDEPTH)
        drain_gather()
        pltpu.async_copy(rows_v.at[b], acc_sh.at[idx_v.at[bi, 1]], ssem,
                         add=True)

        @pl.when(g >= 1)
        def _():
            drain_scatter()

        @pl.when(g + DEPTH < nch)
        def _():
            gi = g + DEPTH
            pltpu.async_copy(ei_hbm.at[s, gi], idx_v.at[lax.rem(gi, IDEPTH)],
                             isem)

        @pl.when(g + DEPTH - 1 < nch)
        def _():
            gn = g + DEPTH - 1
            drain_idx()
            pltpu.async_copy(hp_hbm.at[c].at[idx_v.at[lax.rem(gn, IDEPTH), 0]],
                             rows_v.at[lax.rem(gn, DEPTH)], gsem)

        return carry

    lax.fori_loop(0, nch, body, 0, unroll=False)
    drain_scatter()
    plsc.subcore_barrier()
    pltpu.sync_copy(acc_sh.at[pl.ds(s * ROWS_PER_TILE, ROWS_PER_TILE)],
                    raw_out.at[c, pl.ds(s * ROWS_PER_TILE, ROWS_PER_TILE)])

    @pl.when(s == 0)
    def _():
        pltpu.sync_copy(acc_sh.at[pl.ds(NS * ROWS_PER_TILE, TAIL_ROWS)],
                        raw_out.at[c, pl.ds(NS * ROWS_PER_TILE, TAIL_ROWS)])


def _agg_call(hp, ei_c):
    mesh = plsc.VectorSubcoreMesh(core_axis_name="c", subcore_axis_name="s")
    return pl.kernel(
        _agg_body,
        out_type=jax.ShapeDtypeStruct((NC, N, DH), jnp.float32),
        mesh=mesh,
        scratch_types=[
            pltpu.VMEM((IDEPTH, 2, K), jnp.int32),
            pltpu.VMEM((DEPTH, K, DH), jnp.float32),
            pltpu.VMEM_SHARED((N, DH), jnp.float32),
            pltpu.SemaphoreType.DMA,
            pltpu.SemaphoreType.DMA,
            pltpu.SemaphoreType.DMA,
            pltpu.SemaphoreType.DMA,
        ],
    )(hp, ei_c)


def _dis_from(deg_ref):
    dval = deg_ref[...]
    d = dval[0, 0, :] + dval[0, 1, :] + 1.0
    return lax.rsqrt(d)


def _mmscale_body(x_ref, w_ref, deg_ref, hp_ref):
    dis = _dis_from(deg_ref)
    h = jnp.dot(x_ref[...], w_ref[...], preferred_element_type=jnp.float32)
    hs = h * dis[:, None]
    hp_ref[0] = hs[:, :DH]
    hp_ref[1] = hs[:, DH:]


def _mmscale_call(x, W, deg_r):
    return pl.pallas_call(
        _mmscale_body,
        grid=(GRID,),
        in_specs=[
            pl.BlockSpec((RB, D), lambda i: (i, 0)),
            pl.BlockSpec((D, D), lambda i: (0, 0)),
            pl.BlockSpec((1, NC, RB), lambda i: (i, 0, 0)),
        ],
        out_specs=pl.BlockSpec((NC, RB, DH), lambda i: (0, i, 0)),
        out_shape=jax.ShapeDtypeStruct((NC, N, DH), jnp.float32),
    )(x, W, deg_r)


def _final_body(raw_ref, deg_ref, b_ref, o_ref):
    dis = _dis_from(deg_ref)
    r = jnp.concatenate([raw_ref[0], raw_ref[1]], axis=-1)
    o_ref[...] = dis[:, None] * r + b_ref[...]


def _final_call(raw, deg_r, b2):
    return pl.pallas_call(
        _final_body,
        grid=(GRID,),
        in_specs=[
            pl.BlockSpec((NC, RB, DH), lambda i: (0, i, 0)),
            pl.BlockSpec((1, NC, RB), lambda i: (i, 0, 0)),
            pl.BlockSpec((1, D), lambda i: (0, 0)),
        ],
        out_specs=pl.BlockSpec((RB, D), lambda i: (i, 0)),
        out_shape=jax.ShapeDtypeStruct((N, D), jnp.float32),
    )(raw, deg_r, b2)


def kernel(x, edge_index, W, b):
    src = edge_index[0].astype(jnp.int32)
    dst = edge_index[1].astype(jnp.int32)
    dst_a = dst.reshape(NC, NS, E // (NC * NS * KD), KD)
    nch = E // (NS * K)
    ei_c = jnp.stack([src.reshape(NS, nch, K), dst.reshape(NS, nch, K)],
                     axis=2)
    ones = jnp.ones((KD,), jnp.float32)
    zeros = jnp.zeros((N,), jnp.float32)

    deg_p = _deg_call(dst_a, ones, zeros)
    deg_r = deg_p.reshape(NC, GRID, RB).transpose(1, 0, 2)
    hp = _mmscale_call(x, W, deg_r)
    raw = _agg_call(hp, ei_c)
    return _final_call(raw, deg_r, b.reshape(1, D))
